# baseline (speedup 1.0000x reference)
import functools

import jax
import jax.numpy as jnp
from jax import lax
from jax.experimental import pallas as pl
from jax.experimental.pallas import tpu as pltpu
from jax.experimental.pallas import tpu_sc as plsc

_NC, _NS = 2, 16
_NW = _NC * _NS
_EL = 128


def _leaky(v):
    return jnp.where(v >= 0, v, jnp.float32(0.1) * v)


def _sc_mesh():
    return plsc.VectorSubcoreMesh(core_axis_name="c", subcore_axis_name="s",
                                  num_cores=_NC, num_subcores=_NS)


def _hist_kernel(TAB, R):
    HALF = TAB // 2

    @functools.partial(
        pl.kernel,
        out_type=jax.ShapeDtypeStruct((_NC, _NS, 2, 16, HALF), jnp.float32),
        mesh=_sc_mesh(),
        compiler_params=pltpu.CompilerParams(needs_layout_passes=False),
        scratch_types=[
            pltpu.VMEM((R, _EL), jnp.int32),
            pltpu.VMEM((R, _EL), jnp.int32),
            pltpu.VMEM((R, _EL), jnp.int32),
            pltpu.VMEM((R, _EL), jnp.int32),
            pltpu.VMEM((16,), jnp.int32),
            pltpu.VMEM((16, HALF), jnp.float32),
        ],
    )
    def k(src_hbm, dst_hbm, cur_hbm, zh_hbm, hist_out,
          srcA, srcB, dstA, dstB, cur_v, hist_v):
        c = lax.axis_index("c")
        s = lax.axis_index("s")
        pltpu.sync_copy(src_hbm.at[s * 2], srcA)
        pltpu.sync_copy(src_hbm.at[s * 2 + 1], srcB)
        pltpu.sync_copy(dst_hbm.at[s * 2], dstA)
        pltpu.sync_copy(dst_hbm.at[s * 2 + 1], dstB)
        pltpu.sync_copy(cur_hbm, cur_v)

        curv = cur_v[...]
        ones = jnp.ones((16,), jnp.float32)
        lane16 = lax.iota(jnp.int32, 16)
        lo = c * HALF

        for p in range(2):
            pltpu.sync_copy(zh_hbm, hist_v)
            for sv_ref, dv_ref in ((srcA, dstA), (srcB, dstB)):

                @pl.loop(0, R)
                def _(j):
                    for c8 in range(_EL // 16):
                        d16 = dv_ref[j, pl.ds(c8 * 16, 16)]
                        rel = d16 - lo
                        m = (rel >= 0) & (rel < HALF)
                        if p == 1:
                            s16 = sv_ref[j, pl.ds(c8 * 16, 16)]
                            m = m & (s16 == curv)
                        plsc.addupdate_scatter(hist_v, [lane16, rel], ones,
                                               mask=m)

            pltpu.sync_copy(hist_v, hist_out.at[c].at[s].at[p])

    return k


def _agg_kernel(TAB, R):
    stripe = TAB // _NS

    H1 = (R + 1) // 2

    @functools.partial(
        pl.kernel,
        out_type=jax.ShapeDtypeStruct((_NC, TAB, 128), jnp.float32),
        mesh=_sc_mesh(),
        scratch_types=[
            pltpu.VMEM((H1, _EL), jnp.int32),
            pltpu.VMEM((H1, _EL), jnp.int32),
            pltpu.VMEM((_EL, 128), jnp.float32),
            pltpu.VMEM((_EL, 128), jnp.float32),
            pltpu.SemaphoreType.DMA,
            pltpu.SemaphoreType.DMA,
            pltpu.VMEM_SHARED((TAB, 128), jnp.float32),
        ],
    )
    def k(xs_hbm, src_hbm, dst_hbm, z128_hbm, agg_out,
          src_v, dst_v, rb0, rb1, sem0, sem1, agg_t):
        c = lax.axis_index("c")
        s = lax.axis_index("s")
        wid = s * _NC + c
        pltpu.sync_copy(z128_hbm, agg_t.at[pl.ds(s * stripe, stripe)])
        plsc.subcore_barrier()

        def gather(j, rb, sem):
            return pltpu.async_copy(xs_hbm.at[src_v.at[j]], rb, sem)

        def wait(j, rb, sem):
            pltpu.make_async_copy(xs_hbm.at[src_v.at[j]], rb, sem).wait()

        def scatter(j, rb):
            pltpu.sync_copy(rb, agg_t.at[dst_v.at[j]], add=True)

        for h in range(2):
            lo = h * H1
            cnt = H1 if h == 0 else R - H1
            pltpu.sync_copy(src_hbm.at[wid].at[pl.ds(lo, cnt)],
                            src_v.at[pl.ds(0, cnt)])
            pltpu.sync_copy(dst_hbm.at[wid].at[pl.ds(lo, cnt)],
                            dst_v.at[pl.ds(0, cnt)])
            base = 0
            if cnt % 2 == 0:
                gather(0, rb0, sem0)
                wait(0, rb0, sem0)
                scatter(0, rb0)
                base = 1
            gather(base, rb0, sem0)

            @pl.loop(0, (cnt - base - 1) // 2)
            def _(t):
                j0 = base + 2 * t
                gather(j0 + 1, rb1, sem1)
                wait(j0, rb0, sem0)
                scatter(j0, rb0)
                gather(j0 + 2, rb0, sem0)
                wait(j0 + 1, rb1, sem1)
                scatter(j0 + 1, rb1)

            wait(cnt - 1, rb0, sem0)
            scatter(cnt - 1, rb0)

        plsc.subcore_barrier()
        pltpu.sync_copy(agg_t.at[pl.ds(s * stripe, stripe)],
                        agg_out.at[c].at[pl.ds(s * stripe, stripe)])

    return k


def _histred_body(hist_ref, red_ref):
    i = pl.program_id(0)
    t = hist_ref[0, :, 0, :, :]
    red_ref[pl.ds(i, 1), :] = jnp.sum(t, axis=(0, 1), keepdims=False)[None, :]


def _prep_body(N, TAB, x_ref, deg_ref, xs_ref, dvb_ref):
    x = x_ref[...]
    deg = deg_ref[0:N, :] + jnp.float32(1.0)
    dv = lax.rsqrt(deg)
    xs_ref[0:N, :] = x * dv
    xs_ref[N:TAB, :] = jnp.zeros((TAB - N, 128), jnp.float32)
    dvb_ref[0:N, :] = jnp.broadcast_to(dv, (N, 128))
    dvb_ref[N:TAB, :] = jnp.zeros((TAB - N, 128), jnp.float32)


def _scores_body(RB, D,
                 cur_ref, x_ref, Wp1_ref, bp1_ref, Wp2_ref, bp2_ref,
                 Wp3_ref, bp3_ref, sc_ref):
    i = pl.program_id(0)
    r0 = i * RB
    xb = x_ref[pl.ds(r0, RB), :]
    f32 = jnp.float32
    emb = jnp.tanh(xb)
    cur = cur_ref[0]
    curemb = jnp.tanh(x_ref[pl.ds(cur, 1), :])
    base = jnp.dot(curemb, Wp1_ref[pl.ds(D, D), :],
                   preferred_element_type=f32, precision=lax.Precision.DEFAULT) + bp1_ref[...]
    h1 = _leaky(jnp.dot(emb, Wp1_ref[pl.ds(0, D), :],
                        preferred_element_type=f32, precision=lax.Precision.DEFAULT) + base)
    h2 = _leaky(jnp.dot(h1, Wp2_ref[...], preferred_element_type=f32, precision=lax.Precision.DEFAULT) + bp2_ref[...])
    sc_ref[...] = jnp.dot(h2, Wp3_ref[...], preferred_element_type=f32, precision=lax.Precision.DEFAULT) + bp3_ref[...]


def _critic_body(RB, G,
                 x_ref, dvb_ref, agg_ref, nb_ref, sc_ref,
                 Wc_ref, bc_ref, Wl1_ref, bl1_ref, Wl2_ref, bl2_ref,
                 Wl3_ref, bl3_ref, Wo_ref, bo_ref,
                 probs_ref, val_ref, logit_sc):
    i = pl.program_id(0)
    f32 = jnp.float32

    @pl.when(i < G)
    def _():
        r0 = i * RB
        xb = x_ref[pl.ds(r0, RB), :]
        dv = dvb_ref[pl.ds(r0, RB), :]
        a = agg_ref[0, pl.ds(r0, RB), :] + agg_ref[1, pl.ds(r0, RB), :]
        nb = nb_ref[pl.ds(r0, RB), :]
        aggf = dv * a + dv * dv * xb

        cmat = jnp.dot(aggf, Wc_ref[...], preferred_element_type=f32, precision=lax.Precision.DEFAULT) + bc_ref[...]
        c1 = _leaky(jnp.dot(cmat, Wl1_ref[...], preferred_element_type=f32, precision=lax.Precision.DEFAULT) + bl1_ref[...])
        c2 = _leaky(jnp.dot(c1, Wl2_ref[...], preferred_element_type=f32, precision=lax.Precision.DEFAULT) + bl2_ref[...])
        c3 = _leaky(jnp.dot(c2, Wl3_ref[...], preferred_element_type=f32, precision=lax.Precision.DEFAULT) + bl3_ref[...])
        val_ref[pl.ds(r0, RB), :] = (jnp.dot(c3, Wo_ref[...],
                                             preferred_element_type=f32, precision=lax.Precision.DEFAULT)
                                     + bo_ref[...])
        logit_sc[pl.ds(r0, RB), :] = jnp.where(
            nb > f32(0.5), sc_ref[pl.ds(r0, RB), :], f32(-1e9))

    @pl.when(i == G)
    def _():
        l = logit_sc[...]
        m = jnp.max(l)
        e = jnp.exp(l - m)
        probs_ref[...] = e / jnp.sum(e)


def kernel(x, edge_index, current_vertex_idx, W1, b1, W2, b2, W3, b3,
           Wp1, bp1, Wp2, bp2, Wp3, bp3, Wc, bc, Wl1, bl1, Wl2, bl2,
           Wl3, bl3, Wo, bo):
    N, D = x.shape
    E = edge_index.shape[1]
    R = -(-E // (_NW * _EL))
    EP = _NW * _EL * R
    TAB = -(-(N + 64) // (_NS * 16)) * (_NS * 16)
    stripe = TAB // _NS
    HALF = TAB // 2

    src = edge_index[0]
    dst = edge_index[1]
    padn = EP - E
    pad = (N + (jnp.arange(padn, dtype=jnp.int32) % 64)).astype(jnp.int32)
    srcb = jnp.concatenate([src, pad]).reshape(_NW, R, _EL)
    dstb = jnp.concatenate([dst, pad]).reshape(_NW, R, _EL)

    cur = jnp.asarray(current_vertex_idx, jnp.int32)
    cur1 = cur.reshape((1,))
    cur16 = jnp.full((16,), cur, jnp.int32)
    zh = jnp.zeros((16, HALF), jnp.float32)
    z4 = jnp.zeros((4, HALF), jnp.float32)
    z128 = jnp.zeros((stripe, 128), jnp.float32)
    rowid = (jnp.arange(4, dtype=jnp.int32)[:, None] * 16
             + jnp.arange(16, dtype=jnp.int32)[None, :])

    hist2 = _hist_kernel(TAB, R)(srcb, dstb, cur16, zh)

    red = pl.pallas_call(
        _histred_body,
        grid=(4,),
        in_specs=[pl.BlockSpec((1, _NS, 1, 16, HALF),
                               lambda i: (i % 2, 0, i // 2, 0, 0))],
        out_specs=pl.BlockSpec((4, HALF), lambda i: (0, 0)),
        out_shape=jax.ShapeDtypeStruct((4, HALF), jnp.float32),
    )(hist2)

    deg_col = red[0:2].reshape(TAB, 1)
    nb_col = red[2:4].reshape(TAB, 1)

    xs, dvb = pl.pallas_call(
        functools.partial(_prep_body, N, TAB),
        in_specs=[pl.BlockSpec(x.shape, lambda: (0, 0)),
                  pl.BlockSpec((TAB, 1), lambda: (0, 0))],
        out_specs=(pl.BlockSpec((TAB, 128), lambda: (0, 0)),
                   pl.BlockSpec((TAB, 128), lambda: (0, 0))),
        out_shape=(jax.ShapeDtypeStruct((TAB, 128), jnp.float32),
                   jax.ShapeDtypeStruct((TAB, 128), jnp.float32)),
    )(x, deg_col)

    agg2 = _agg_kernel(TAB, R)(xs, srcb, dstb, z128)

    RB = 2000 if N % 2000 == 0 else (1000 if N % 1000 == 0 else 8)
    G = N // RB
    full = lambda arr: pl.BlockSpec(arr.shape, lambda i: (0,) * arr.ndim)

    def wspecs(ws):
        return [w for w in ws], [pl.BlockSpec(w.shape, lambda i: (0, 0))
                                 for w in ws]

    wp_args, wp_specs = wspecs((Wp1, bp1.reshape(1, -1), Wp2,
                                bp2.reshape(1, -1), Wp3, bp3.reshape(1, -1)))
    scores = pl.pallas_call(
        functools.partial(_scores_body, RB, D),
        grid=(G,),
        in_specs=[pl.BlockSpec(memory_space=pltpu.SMEM), full(x)] + wp_specs,
        out_specs=pl.BlockSpec((RB, 1), lambda i: (i, 0)),
        out_shape=jax.ShapeDtypeStruct((N, 1), jnp.float32),
    )(cur1, x, *wp_args)

    wc_args, wc_specs = wspecs((Wc, bc.reshape(1, -1), Wl1,
                                bl1.reshape(1, -1), Wl2, bl2.reshape(1, -1),
                                Wl3, bl3.reshape(1, -1), Wo,
                                bo.reshape(1, -1)))
    probs, value = pl.pallas_call(
        functools.partial(_critic_body, RB, G),
        grid=(G + 1,),
        in_specs=[full(x), full(dvb), full(agg2), full(nb_col),
                  full(scores)] + wc_specs,
        out_specs=(pl.BlockSpec((N, 1), lambda i: (0, 0)),
                   pl.BlockSpec((N, 1), lambda i: (0, 0))),
        out_shape=(jax.ShapeDtypeStruct((N, 1), jnp.float32),
                   jax.ShapeDtypeStruct((N, 1), jnp.float32)),
        scratch_shapes=[pltpu.VMEM((N, 1), jnp.float32)],
    )(x, dvb, agg2, nb_col, scores, *wc_args)

    return probs[:, 0], value


# speedup vs baseline: 1.0090x; 1.0090x over previous
import functools

import jax
import jax.numpy as jnp
from jax import lax
from jax.experimental import pallas as pl
from jax.experimental.pallas import tpu as pltpu
from jax.experimental.pallas import tpu_sc as plsc

_NC, _NS = 2, 16
_NW = _NC * _NS
_EL = 128


def _leaky(v):
    return jnp.where(v >= 0, v, jnp.float32(0.1) * v)


def _sc_mesh():
    return plsc.VectorSubcoreMesh(core_axis_name="c", subcore_axis_name="s",
                                  num_cores=_NC, num_subcores=_NS)


def _hist_kernel(TAB, R):
    HALF = TAB // 2

    @functools.partial(
        pl.kernel,
        out_type=jax.ShapeDtypeStruct((_NC, _NS, 2, 16, HALF), jnp.float32),
        mesh=_sc_mesh(),
        compiler_params=pltpu.CompilerParams(needs_layout_passes=False),
        scratch_types=[
            pltpu.VMEM((R, _EL), jnp.int32),
            pltpu.VMEM((R, _EL), jnp.int32),
            pltpu.VMEM((R, _EL), jnp.int32),
            pltpu.VMEM((R, _EL), jnp.int32),
            pltpu.VMEM((16,), jnp.int32),
            pltpu.VMEM((16, HALF), jnp.float32),
        ],
    )
    def k(src_hbm, dst_hbm, cur_hbm, zh_hbm, hist_out,
          srcA, srcB, dstA, dstB, cur_v, hist_v):
        c = lax.axis_index("c")
        s = lax.axis_index("s")
        pltpu.sync_copy(src_hbm.at[s * 2], srcA)
        pltpu.sync_copy(src_hbm.at[s * 2 + 1], srcB)
        pltpu.sync_copy(dst_hbm.at[s * 2], dstA)
        pltpu.sync_copy(dst_hbm.at[s * 2 + 1], dstB)
        pltpu.sync_copy(cur_hbm, cur_v)

        curv = cur_v[...]
        ones = jnp.ones((16,), jnp.float32)
        lane16 = lax.iota(jnp.int32, 16)
        lo = c * HALF

        for p in range(2):
            pltpu.sync_copy(zh_hbm, hist_v)
            for sv_ref, dv_ref in ((srcA, dstA), (srcB, dstB)):

                @pl.loop(0, R)
                def _(j):
                    for c8 in range(_EL // 16):
                        d16 = dv_ref[j, pl.ds(c8 * 16, 16)]
                        rel = d16 - lo
                        m = (rel >= 0) & (rel < HALF)
                        if p == 1:
                            s16 = sv_ref[j, pl.ds(c8 * 16, 16)]
                            m = m & (s16 == curv)
                        plsc.addupdate_scatter(hist_v, [lane16, rel], ones,
                                               mask=m)

            pltpu.sync_copy(hist_v, hist_out.at[c].at[s].at[p])

    return k


def _agg_kernel(TAB, R):
    stripe = TAB // _NS

    H1 = (R + 1) // 2

    @functools.partial(
        pl.kernel,
        out_type=jax.ShapeDtypeStruct((_NC, TAB, 128), jnp.float32),
        mesh=_sc_mesh(),
        scratch_types=[
            pltpu.VMEM((H1, _EL), jnp.int32),
            pltpu.VMEM((H1, _EL), jnp.int32),
            pltpu.VMEM((_EL, 128), jnp.float32),
            pltpu.VMEM((_EL, 128), jnp.float32),
            pltpu.SemaphoreType.DMA,
            pltpu.SemaphoreType.DMA,
            pltpu.VMEM_SHARED((TAB, 128), jnp.float32),
        ],
    )
    def k(xs_hbm, src_hbm, dst_hbm, z128_hbm, agg_out,
          src_v, dst_v, rb0, rb1, sem0, sem1, agg_t):
        c = lax.axis_index("c")
        s = lax.axis_index("s")
        wid = s * _NC + c
        pltpu.sync_copy(z128_hbm, agg_t.at[pl.ds(s * stripe, stripe)])
        plsc.subcore_barrier()

        def gather(j, rb, sem):
            return pltpu.async_copy(xs_hbm.at[src_v.at[j]], rb, sem)

        def wait(j, rb, sem):
            pltpu.make_async_copy(xs_hbm.at[src_v.at[j]], rb, sem).wait()

        def scatter(j, rb):
            pltpu.sync_copy(rb, agg_t.at[dst_v.at[j]], add=True)

        for h in range(2):
            lo = h * H1
            cnt = H1 if h == 0 else R - H1
            pltpu.sync_copy(src_hbm.at[wid].at[pl.ds(lo, cnt)],
                            src_v.at[pl.ds(0, cnt)])
            pltpu.sync_copy(dst_hbm.at[wid].at[pl.ds(lo, cnt)],
                            dst_v.at[pl.ds(0, cnt)])
            base = 0
            if cnt % 2 == 0:
                gather(0, rb0, sem0)
                wait(0, rb0, sem0)
                scatter(0, rb0)
                base = 1
            gather(base, rb0, sem0)

            @pl.loop(0, (cnt - base - 1) // 2)
            def _(t):
                j0 = base + 2 * t
                gather(j0 + 1, rb1, sem1)
                wait(j0, rb0, sem0)
                scatter(j0, rb0)
                gather(j0 + 2, rb0, sem0)
                wait(j0 + 1, rb1, sem1)
                scatter(j0 + 1, rb1)

            wait(cnt - 1, rb0, sem0)
            scatter(cnt - 1, rb0)

        plsc.subcore_barrier()
        pltpu.sync_copy(agg_t.at[pl.ds(s * stripe, stripe)],
                        agg_out.at[c].at[pl.ds(s * stripe, stripe)])

    return k


def _histred_body(hist_ref, red_ref):
    i = pl.program_id(0)
    t = hist_ref[0, :, 0, :, :]
    red_ref[pl.ds(i, 1), :] = jnp.sum(t, axis=(0, 1), keepdims=False)[None, :]


def _prep_body(N, TAB, x_ref, deg_ref, xs_ref):
    x = x_ref[...]
    deg = deg_ref[0:N, :] + jnp.float32(1.0)
    dv = lax.rsqrt(deg)
    xs_ref[0:N, :] = x * dv
    xs_ref[N:TAB, :] = jnp.zeros((TAB - N, 128), jnp.float32)


def _scores_body(RB, D,
                 cur_ref, x_ref, Wp1_ref, bp1_ref, Wp2_ref, bp2_ref,
                 Wp3_ref, bp3_ref, sc_ref):
    i = pl.program_id(0)
    r0 = i * RB
    xb = x_ref[pl.ds(r0, RB), :]
    f32 = jnp.float32
    emb = jnp.tanh(xb)
    cur = cur_ref[0]
    curemb = jnp.tanh(x_ref[pl.ds(cur, 1), :])
    base = jnp.dot(curemb, Wp1_ref[pl.ds(D, D), :],
                   preferred_element_type=f32, precision=lax.Precision.DEFAULT) + bp1_ref[...]
    h1 = _leaky(jnp.dot(emb, Wp1_ref[pl.ds(0, D), :],
                        preferred_element_type=f32, precision=lax.Precision.DEFAULT) + base)
    h2 = _leaky(jnp.dot(h1, Wp2_ref[...], preferred_element_type=f32, precision=lax.Precision.DEFAULT) + bp2_ref[...])
    sc_ref[...] = jnp.dot(h2, Wp3_ref[...], preferred_element_type=f32, precision=lax.Precision.DEFAULT) + bp3_ref[...]


def _critic_body(RB, G,
                 x_ref, deg_ref, agg_ref, nb_ref, sc_ref,
                 Wc_ref, bc_ref, Wl1_ref, bl1_ref, Wl2_ref, bl2_ref,
                 Wl3_ref, bl3_ref, Wo_ref, bo_ref,
                 probs_ref, val_ref, logit_sc):
    i = pl.program_id(0)
    f32 = jnp.float32

    @pl.when(i < G)
    def _():
        r0 = i * RB
        xb = x_ref[pl.ds(r0, RB), :]
        dv = lax.rsqrt(deg_ref[pl.ds(r0, RB), :] + f32(1.0))
        a = agg_ref[0, pl.ds(r0, RB), :] + agg_ref[1, pl.ds(r0, RB), :]
        nb = nb_ref[pl.ds(r0, RB), :]
        aggf = dv * a + dv * dv * xb

        cmat = jnp.dot(aggf, Wc_ref[...], preferred_element_type=f32, precision=lax.Precision.DEFAULT) + bc_ref[...]
        c1 = _leaky(jnp.dot(cmat, Wl1_ref[...], preferred_element_type=f32, precision=lax.Precision.DEFAULT) + bl1_ref[...])
        c2 = _leaky(jnp.dot(c1, Wl2_ref[...], preferred_element_type=f32, precision=lax.Precision.DEFAULT) + bl2_ref[...])
        c3 = _leaky(jnp.dot(c2, Wl3_ref[...], preferred_element_type=f32, precision=lax.Precision.DEFAULT) + bl3_ref[...])
        val_ref[pl.ds(r0, RB), :] = (jnp.dot(c3, Wo_ref[...],
                                             preferred_element_type=f32, precision=lax.Precision.DEFAULT)
                                     + bo_ref[...])
        logit_sc[pl.ds(r0, RB), :] = jnp.where(
            nb > f32(0.5), sc_ref[pl.ds(r0, RB), :], f32(-1e9))

    @pl.when(i == G)
    def _():
        l = logit_sc[...]
        m = jnp.max(l)
        e = jnp.exp(l - m)
        probs_ref[...] = e / jnp.sum(e)


def kernel(x, edge_index, current_vertex_idx, W1, b1, W2, b2, W3, b3,
           Wp1, bp1, Wp2, bp2, Wp3, bp3, Wc, bc, Wl1, bl1, Wl2, bl2,
           Wl3, bl3, Wo, bo):
    N, D = x.shape
    E = edge_index.shape[1]
    R = -(-E // (_NW * _EL))
    EP = _NW * _EL * R
    TAB = -(-(N + 64) // (_NS * 16)) * (_NS * 16)
    stripe = TAB // _NS
    HALF = TAB // 2

    src = edge_index[0]
    dst = edge_index[1]
    padn = EP - E
    pad = (N + (jnp.arange(padn, dtype=jnp.int32) % 64)).astype(jnp.int32)
    srcb = jnp.concatenate([src, pad]).reshape(_NW, R, _EL)
    dstb = jnp.concatenate([dst, pad]).reshape(_NW, R, _EL)

    cur = jnp.asarray(current_vertex_idx, jnp.int32)
    cur1 = cur.reshape((1,))
    cur16 = jnp.full((16,), cur, jnp.int32)
    zh = jnp.zeros((16, HALF), jnp.float32)
    z4 = jnp.zeros((4, HALF), jnp.float32)
    z128 = jnp.zeros((stripe, 128), jnp.float32)
    rowid = (jnp.arange(4, dtype=jnp.int32)[:, None] * 16
             + jnp.arange(16, dtype=jnp.int32)[None, :])

    hist2 = _hist_kernel(TAB, R)(srcb, dstb, cur16, zh)

    red = pl.pallas_call(
        _histred_body,
        grid=(4,),
        in_specs=[pl.BlockSpec((1, _NS, 1, 16, HALF),
                               lambda i: (i % 2, 0, i // 2, 0, 0))],
        out_specs=pl.BlockSpec((4, HALF), lambda i: (0, 0)),
        out_shape=jax.ShapeDtypeStruct((4, HALF), jnp.float32),
    )(hist2)

    deg_col = red[0:2].reshape(TAB, 1)
    nb_col = red[2:4].reshape(TAB, 1)

    xs = pl.pallas_call(
        functools.partial(_prep_body, N, TAB),
        in_specs=[pl.BlockSpec(x.shape, lambda: (0, 0)),
                  pl.BlockSpec((TAB, 1), lambda: (0, 0))],
        out_specs=pl.BlockSpec((TAB, 128), lambda: (0, 0)),
        out_shape=jax.ShapeDtypeStruct((TAB, 128), jnp.float32),
    )(x, deg_col)

    agg2 = _agg_kernel(TAB, R)(xs, srcb, dstb, z128)

    RB = 2000 if N % 2000 == 0 else (1000 if N % 1000 == 0 else 8)
    G = N // RB
    full = lambda arr: pl.BlockSpec(arr.shape, lambda i: (0,) * arr.ndim)

    def wspecs(ws):
        return [w for w in ws], [pl.BlockSpec(w.shape, lambda i: (0, 0))
                                 for w in ws]

    wp_args, wp_specs = wspecs((Wp1, bp1.reshape(1, -1), Wp2,
                                bp2.reshape(1, -1), Wp3, bp3.reshape(1, -1)))
    scores = pl.pallas_call(
        functools.partial(_scores_body, RB, D),
        grid=(G,),
        in_specs=[pl.BlockSpec(memory_space=pltpu.SMEM), full(x)] + wp_specs,
        out_specs=pl.BlockSpec((RB, 1), lambda i: (i, 0)),
        out_shape=jax.ShapeDtypeStruct((N, 1), jnp.float32),
    )(cur1, x, *wp_args)

    wc_args, wc_specs = wspecs((Wc, bc.reshape(1, -1), Wl1,
                                bl1.reshape(1, -1), Wl2, bl2.reshape(1, -1),
                                Wl3, bl3.reshape(1, -1), Wo,
                                bo.reshape(1, -1)))
    probs, value = pl.pallas_call(
        functools.partial(_critic_body, RB, G),
        grid=(G + 1,),
        in_specs=[full(x), full(deg_col), full(agg2), full(nb_col),
                  full(scores)] + wc_specs,
        out_specs=(pl.BlockSpec((N, 1), lambda i: (0, 0)),
                   pl.BlockSpec((N, 1), lambda i: (0, 0))),
        out_shape=(jax.ShapeDtypeStruct((N, 1), jnp.float32),
                   jax.ShapeDtypeStruct((N, 1), jnp.float32)),
        scratch_shapes=[pltpu.VMEM((N, 1), jnp.float32)],
    )(x, deg_col, agg2, nb_col, scores, *wc_args)

    return probs[:, 0], value
